# baseline (speedup 1.0000x reference)
import functools

import jax
import jax.numpy as jnp
from jax.experimental import pallas as pl

_BM = 512


def _router_block(x_ref, w_ref, o_ref):
    x = x_ref[...]
    w = w_ref[...]
    logits = jax.lax.dot_general(
        x, w,
        dimension_numbers=(((1,), (1,)), ((), ())),
        preferred_element_type=jnp.float32,
    )
    m = jnp.max(logits, axis=-1, keepdims=True)
    e = jnp.exp(logits - m)
    o_ref[...] = e / jnp.sum(e, axis=-1, keepdims=True)


@functools.partial(jax.jit, static_argnames=())
def kernel(inputs, W):
    M, K = inputs.shape
    E = W.shape[0]
    grid = (M // _BM,)
    return pl.pallas_call(
        _router_block,
        grid=grid,
        in_specs=[
            pl.BlockSpec((_BM, K), lambda i: (i, 0)),
            pl.BlockSpec((E, K), lambda i: (0, 0)),
        ],
        out_specs=pl.BlockSpec((_BM, E), lambda i: (i, 0)),
        out_shape=jax.ShapeDtypeStruct((M, E), jnp.float32),
    )(inputs, W)


# speedup vs baseline: 1.1788x; 1.1788x over previous
import functools

import jax
import jax.numpy as jnp
from jax.experimental import pallas as pl

_BM = 1024


def _router_block(x_ref, w_ref, o_ref):
    x = x_ref[...]
    w = w_ref[...]
    logits = jax.lax.dot_general(
        x, w,
        dimension_numbers=(((1,), (1,)), ((), ())),
        preferred_element_type=jnp.float32,
    )
    m = jnp.max(logits, axis=-1, keepdims=True)
    e = jnp.exp(logits - m)
    o_ref[...] = e / jnp.sum(e, axis=-1, keepdims=True)


@functools.partial(jax.jit, static_argnames=())
def kernel(inputs, W):
    M, K = inputs.shape
    E = W.shape[0]
    grid = (M // _BM,)
    return pl.pallas_call(
        _router_block,
        grid=grid,
        in_specs=[
            pl.BlockSpec((_BM, K), lambda i: (i, 0)),
            pl.BlockSpec((E, K), lambda i: (0, 0)),
        ],
        out_specs=pl.BlockSpec((_BM, E), lambda i: (i, 0)),
        out_shape=jax.ShapeDtypeStruct((M, E), jnp.float32),
    )(inputs, W)
